# baseline (speedup 1.0000x reference)
import functools

import jax
import jax.numpy as jnp
from jax.experimental import pallas as pl
from jax.experimental.pallas import tpu as pltpu


_RB = 128
_CH = 64


def _edges_mask_body(N, ptr_ref, in_ref, clsrow_ref, out_e_ref, out_m_ref):
    b = pl.program_id(0)
    r = pl.program_id(1)
    Ed = clsrow_ref.shape[2] // N
    rows = jax.lax.broadcasted_iota(jnp.int32, (1, _RB, 1), 1) + r * _RB
    is_cls_row = rows == N
    body = jnp.where(is_cls_row, clsrow_ref[...], in_ref[...])
    cls_col = jnp.broadcast_to(clsrow_ref[:, :, 0:Ed], (1, _RB, Ed))
    out_e_ref[...] = jnp.concatenate([body, cls_col], axis=2)

    c = ptr_ref[b + 1] - ptr_ref[b]
    f_rows = (rows < c) | is_cls_row
    cols = jax.lax.broadcasted_iota(jnp.int32, (1, 1, N + 1), 2)
    f_cols = (cols < c) | (cols == N)
    m = f_rows & f_cols & (c > 0)
    out_m_ref[...] = m


def _x_body(B, T, ptr_ref, x_ref, cls_ref, out_x_ref, out_nb_ref, out_cm_ref):
    D = x_ref.shape[1]
    TB = T + B
    x_full = x_ref[...]
    o2 = jax.lax.broadcasted_iota(jnp.int32, (TB, 1), 0)
    acc = jnp.zeros((TB, D), x_full.dtype)
    cls_sel = jnp.zeros((TB, 1), jnp.bool_)
    for b in range(B):
        parts = [x_full, jnp.zeros((B - b, D), x_full.dtype)]
        if b > 0:
            parts.insert(0, jnp.zeros((b, D), x_full.dtype))
        sh = jnp.concatenate(parts, axis=0)
        sel = (o2 >= ptr_ref[b] + b) & (o2 < ptr_ref[b + 1] + b)
        acc = jnp.where(sel, sh, acc)
        cls_sel = cls_sel | (o2 == ptr_ref[b + 1] + b)
    acc = jnp.where(cls_sel, cls_ref[...], acc)
    out_x_ref[...] = acc

    rows, cols = out_nb_ref.shape
    o = (jax.lax.broadcasted_iota(jnp.int32, (rows, cols), 0) * cols
         + jax.lax.broadcasted_iota(jnp.int32, (rows, cols), 1))
    nb = jnp.zeros((rows, cols), jnp.int32)
    cm = jnp.zeros((rows, cols), jnp.bool_)
    for b in range(1, B + 1):
        nb = nb + (o >= ptr_ref[b] + b).astype(jnp.int32)
    for b in range(B):
        cm = cm | (o == ptr_ref[b + 1] + b)
    out_nb_ref[...] = nb
    out_cm_ref[...] = cm


def kernel(x, batch_ids, ptr, edges, cls, cls_edge):
    B, N, _, Ed = edges.shape
    T, D = x.shape
    W = N * Ed

    e2 = edges.reshape(B, N, W)
    clsrow = jnp.broadcast_to(cls_edge.reshape(1, 1, 1, Ed), (1, 1, N, Ed)).reshape(1, 1, W)

    n_rblocks = (N + 1 + _RB - 1) // _RB
    nb_in = N // _RB

    out_e, mask = pl.pallas_call(
        functools.partial(_edges_mask_body, N),
        grid=(B, n_rblocks),
        in_specs=[
            pl.BlockSpec(memory_space=pltpu.SMEM),
            pl.BlockSpec((1, _RB, W), lambda b, r: (b, jnp.minimum(r, nb_in - 1), 0)),
            pl.BlockSpec((1, 1, W), lambda b, r: (0, 0, 0)),
        ],
        out_specs=[
            pl.BlockSpec((1, _RB, W + Ed), lambda b, r: (b, r, 0)),
            pl.BlockSpec((1, _RB, N + 1), lambda b, r: (b, r, 0)),
        ],
        out_shape=[
            jax.ShapeDtypeStruct((B, N + 1, W + Ed), jnp.float32),
            jax.ShapeDtypeStruct((B, N + 1, N + 1), jnp.bool_),
        ],
    )(ptr, e2, clsrow)

    edges_out = out_e.reshape(B, N + 1, N + 1, Ed)

    TB = T + B
    rows = B
    cols = TB // B
    assert rows * cols == TB

    new_x, nb, cm = pl.pallas_call(
        functools.partial(_x_body, B, T),
        in_specs=[
            pl.BlockSpec(memory_space=pltpu.SMEM),
            pl.BlockSpec(memory_space=pltpu.VMEM),
            pl.BlockSpec(memory_space=pltpu.VMEM),
        ],
        out_specs=[
            pl.BlockSpec(memory_space=pltpu.VMEM),
            pl.BlockSpec(memory_space=pltpu.VMEM),
            pl.BlockSpec(memory_space=pltpu.VMEM),
        ],
        out_shape=[
            jax.ShapeDtypeStruct((TB, D), x.dtype),
            jax.ShapeDtypeStruct((rows, cols), jnp.int32),
            jax.ShapeDtypeStruct((rows, cols), jnp.bool_),
        ],
    )(ptr, x, cls.reshape(1, D))

    new_batch = nb.reshape(TB)
    cls_mask = cm.reshape(TB)
    new_ptr = ptr + jnp.arange(B + 1, dtype=ptr.dtype)
    return new_x, mask, edges_out, cls_mask, new_batch, new_ptr


# speedup vs baseline: 6.4080x; 6.4080x over previous
import functools

import jax
import jax.numpy as jnp
from jax.experimental import pallas as pl
from jax.experimental.pallas import tpu as pltpu


_RB = 512


def _edges_mask_body(N, nfull, ptr_ref, in_ref, clsblk_ref, out_e_ref, out_m_ref):
    b = pl.program_id(0)
    r = pl.program_id(1)

    @pl.when(r < nfull)
    def _():
        out_e_ref[...] = jnp.concatenate(
            [in_ref[...], clsblk_ref[:, :, N:N + 1]], axis=2)

    @pl.when(r >= nfull)
    def _():
        out_e_ref[...] = clsblk_ref[...]

    @pl.when(r == 0)
    def _():
        c = ptr_ref[b + 1] - ptr_ref[b]
        rows = jax.lax.broadcasted_iota(jnp.int32, (1, N + 1, 1), 1)
        cols = jax.lax.broadcasted_iota(jnp.int32, (1, 1, N + 1), 2)
        f_rows = (rows < c) | (rows == N)
        f_cols = (cols < c) | (cols == N)
        out_m_ref[...] = f_rows & f_cols & (c > 0)


def _x_body(B, T, ptr_ref, x_ref, cls_ref, out_x_ref, out_nb_ref, out_cm_ref):
    D = x_ref.shape[1]
    TB = T + B
    x_full = x_ref[...]
    o2 = jax.lax.broadcasted_iota(jnp.int32, (TB, 1), 0)
    acc = jnp.zeros((TB, D), x_full.dtype)
    cls_sel = jnp.zeros((TB, 1), jnp.bool_)
    for b in range(B):
        parts = [x_full, jnp.zeros((B - b, D), x_full.dtype)]
        if b > 0:
            parts.insert(0, jnp.zeros((b, D), x_full.dtype))
        sh = jnp.concatenate(parts, axis=0)
        sel = (o2 >= ptr_ref[b] + b) & (o2 < ptr_ref[b + 1] + b)
        acc = jnp.where(sel, sh, acc)
        cls_sel = cls_sel | (o2 == ptr_ref[b + 1] + b)
    acc = jnp.where(cls_sel, cls_ref[...], acc)
    out_x_ref[...] = acc

    rows, cols = out_nb_ref.shape
    o = (jax.lax.broadcasted_iota(jnp.int32, (rows, cols), 0) * cols
         + jax.lax.broadcasted_iota(jnp.int32, (rows, cols), 1))
    nb = jnp.zeros((rows, cols), jnp.int32)
    cm = jnp.zeros((rows, cols), jnp.bool_)
    for b in range(1, B + 1):
        nb = nb + (o >= ptr_ref[b] + b).astype(jnp.int32)
    for b in range(B):
        cm = cm | (o == ptr_ref[b + 1] + b)
    out_nb_ref[...] = nb
    out_cm_ref[...] = cm


def kernel(x, batch_ids, ptr, edges, cls, cls_edge):
    B, N, _, Ed = edges.shape
    T, D = x.shape
    R = N * Ed
    R1 = R + Ed

    et = jnp.transpose(edges, (0, 1, 3, 2)).reshape(B, R, N)

    clsblk = jnp.broadcast_to(
        cls_edge.reshape(1, 1, Ed, 1), (1, _RB // Ed, Ed, N + 1)
    ).reshape(1, _RB, N + 1)

    nfull = R // _RB
    n_rblocks = (R1 + _RB - 1) // _RB

    out_e, mask = pl.pallas_call(
        functools.partial(_edges_mask_body, N, nfull),
        grid=(B, n_rblocks),
        in_specs=[
            pl.BlockSpec(memory_space=pltpu.SMEM),
            pl.BlockSpec((1, _RB, N), lambda b, r: (b, jnp.minimum(r, nfull - 1), 0)),
            pl.BlockSpec((1, _RB, N + 1), lambda b, r: (0, 0, 0)),
        ],
        out_specs=[
            pl.BlockSpec((1, _RB, N + 1), lambda b, r: (b, r, 0)),
            pl.BlockSpec((1, N + 1, N + 1), lambda b, r: (b, 0, 0)),
        ],
        out_shape=[
            jax.ShapeDtypeStruct((B, R1, N + 1), jnp.float32),
            jax.ShapeDtypeStruct((B, N + 1, N + 1), jnp.bool_),
        ],
    )(ptr, et, clsblk)

    edges_out = out_e.reshape(B, N + 1, Ed, N + 1).transpose(0, 1, 3, 2)

    TB = T + B
    rows = B
    cols = TB // B
    assert rows * cols == TB

    new_x, nb, cm = pl.pallas_call(
        functools.partial(_x_body, B, T),
        in_specs=[
            pl.BlockSpec(memory_space=pltpu.SMEM),
            pl.BlockSpec(memory_space=pltpu.VMEM),
            pl.BlockSpec(memory_space=pltpu.VMEM),
        ],
        out_specs=[
            pl.BlockSpec(memory_space=pltpu.VMEM),
            pl.BlockSpec(memory_space=pltpu.VMEM),
            pl.BlockSpec(memory_space=pltpu.VMEM),
        ],
        out_shape=[
            jax.ShapeDtypeStruct((TB, D), x.dtype),
            jax.ShapeDtypeStruct((rows, cols), jnp.int32),
            jax.ShapeDtypeStruct((rows, cols), jnp.bool_),
        ],
    )(ptr, x, cls.reshape(1, D))

    new_batch = nb.reshape(TB)
    cls_mask = cm.reshape(TB)
    new_ptr = ptr + jnp.arange(B + 1, dtype=ptr.dtype)
    return new_x, mask, edges_out, cls_mask, new_batch, new_ptr


# speedup vs baseline: 8.0348x; 1.2539x over previous
import functools

import jax
import jax.numpy as jnp
from jax.experimental import pallas as pl
from jax.experimental.pallas import tpu as pltpu


_RB = 1024


def _edges_mask_body(N, nfull, ptr_ref, in_ref, clsblk_ref, out_e_ref, out_m_ref):
    b = pl.program_id(0)
    r = pl.program_id(1)

    @pl.when(r < nfull)
    def _():
        out_e_ref[...] = jnp.concatenate(
            [in_ref[...], clsblk_ref[:, :, N:N + 1]], axis=2)

    @pl.when(r >= nfull)
    def _():
        out_e_ref[...] = clsblk_ref[...]

    @pl.when(r == 0)
    def _():
        c = ptr_ref[b + 1] - ptr_ref[b]
        rows = jax.lax.broadcasted_iota(jnp.int32, (1, N + 1, 1), 1)
        cols = jax.lax.broadcasted_iota(jnp.int32, (1, 1, N + 1), 2)
        f_rows = (rows < c) | (rows == N)
        f_cols = (cols < c) | (cols == N)
        out_m_ref[...] = (f_rows & f_cols & (c > 0)).astype(jnp.int8)


def _x_body(B, T, ptr_ref, x_ref, cls_ref, out_x_ref, out_nb_ref, out_cm_ref):
    D = x_ref.shape[1]
    TB = T + B
    x_full = x_ref[...]
    o2 = jax.lax.broadcasted_iota(jnp.int32, (TB, 1), 0)
    acc = jnp.zeros((TB, D), x_full.dtype)
    cls_sel = jnp.zeros((TB, 1), jnp.bool_)
    for b in range(B):
        parts = [x_full, jnp.zeros((B - b, D), x_full.dtype)]
        if b > 0:
            parts.insert(0, jnp.zeros((b, D), x_full.dtype))
        sh = jnp.concatenate(parts, axis=0)
        sel = (o2 >= ptr_ref[b] + b) & (o2 < ptr_ref[b + 1] + b)
        acc = jnp.where(sel, sh, acc)
        cls_sel = cls_sel | (o2 == ptr_ref[b + 1] + b)
    acc = jnp.where(cls_sel, cls_ref[...], acc)
    out_x_ref[...] = acc

    rows, cols = out_nb_ref.shape
    o = (jax.lax.broadcasted_iota(jnp.int32, (rows, cols), 0) * cols
         + jax.lax.broadcasted_iota(jnp.int32, (rows, cols), 1))
    nb = jnp.zeros((rows, cols), jnp.int32)
    cm = jnp.zeros((rows, cols), jnp.bool_)
    for b in range(1, B + 1):
        nb = nb + (o >= ptr_ref[b] + b).astype(jnp.int32)
    for b in range(B):
        cm = cm | (o == ptr_ref[b + 1] + b)
    out_nb_ref[...] = nb
    out_cm_ref[...] = cm


def kernel(x, batch_ids, ptr, edges, cls, cls_edge):
    B, N, _, Ed = edges.shape
    T, D = x.shape
    R = N * Ed
    R1 = R + Ed

    et = jnp.transpose(edges, (0, 1, 3, 2)).reshape(B, R, N)

    clsblk = jnp.broadcast_to(
        cls_edge.reshape(1, 1, Ed, 1), (1, _RB // Ed, Ed, N + 1)
    ).reshape(1, _RB, N + 1)

    nfull = R // _RB
    n_rblocks = (R1 + _RB - 1) // _RB

    out_e, mask = pl.pallas_call(
        functools.partial(_edges_mask_body, N, nfull),
        grid=(B, n_rblocks),
        in_specs=[
            pl.BlockSpec(memory_space=pltpu.SMEM),
            pl.BlockSpec((1, _RB, N), lambda b, r: (b, jnp.minimum(r, nfull - 1), 0)),
            pl.BlockSpec((1, _RB, N + 1), lambda b, r: (0, 0, 0)),
        ],
        out_specs=[
            pl.BlockSpec((1, _RB, N + 1), lambda b, r: (b, r, 0)),
            pl.BlockSpec((1, N + 1, N + 1), lambda b, r: (b, 0, 0)),
        ],
        out_shape=[
            jax.ShapeDtypeStruct((B, R1, N + 1), jnp.float32),
            jax.ShapeDtypeStruct((B, N + 1, N + 1), jnp.int8),
        ],
    )(ptr, et, clsblk)
    mask = mask.astype(jnp.bool_)

    edges_out = out_e.reshape(B, N + 1, Ed, N + 1).transpose(0, 1, 3, 2)

    TB = T + B
    rows = B
    cols = TB // B
    assert rows * cols == TB

    new_x, nb, cm = pl.pallas_call(
        functools.partial(_x_body, B, T),
        in_specs=[
            pl.BlockSpec(memory_space=pltpu.SMEM),
            pl.BlockSpec(memory_space=pltpu.VMEM),
            pl.BlockSpec(memory_space=pltpu.VMEM),
        ],
        out_specs=[
            pl.BlockSpec(memory_space=pltpu.VMEM),
            pl.BlockSpec(memory_space=pltpu.VMEM),
            pl.BlockSpec(memory_space=pltpu.VMEM),
        ],
        out_shape=[
            jax.ShapeDtypeStruct((TB, D), x.dtype),
            jax.ShapeDtypeStruct((rows, cols), jnp.int32),
            jax.ShapeDtypeStruct((rows, cols), jnp.bool_),
        ],
    )(ptr, x, cls.reshape(1, D))

    new_batch = nb.reshape(TB)
    cls_mask = cm.reshape(TB)
    new_ptr = ptr + jnp.arange(B + 1, dtype=ptr.dtype)
    return new_x, mask, edges_out, cls_mask, new_batch, new_ptr


# speedup vs baseline: 8.5142x; 1.0597x over previous
import functools

import jax
import jax.numpy as jnp
from jax.experimental import pallas as pl
from jax.experimental.pallas import tpu as pltpu


_RB = 2048


def _edges_mask_body(N, nfull, ptr_ref, in_ref, clsblk_ref, out_e_ref, out_m_ref):
    b = pl.program_id(0)
    r = pl.program_id(1)

    @pl.when(r < nfull)
    def _():
        out_e_ref[...] = jnp.concatenate(
            [in_ref[...], clsblk_ref[:, :, N:N + 1]], axis=2)

    @pl.when(r >= nfull)
    def _():
        out_e_ref[...] = clsblk_ref[...]

    @pl.when(r == 0)
    def _():
        c = ptr_ref[b + 1] - ptr_ref[b]
        rows = jax.lax.broadcasted_iota(jnp.int32, (1, N + 1, 1), 1)
        cols = jax.lax.broadcasted_iota(jnp.int32, (1, 1, N + 1), 2)
        f_rows = (rows < c) | (rows == N)
        f_cols = (cols < c) | (cols == N)
        out_m_ref[...] = (f_rows & f_cols & (c > 0)).astype(jnp.int8)


def _x_body(B, T, ptr_ref, x_ref, cls_ref, out_x_ref, out_nb_ref, out_cm_ref):
    D = x_ref.shape[1]
    TB = T + B
    x_full = x_ref[...]
    o2 = jax.lax.broadcasted_iota(jnp.int32, (TB, 1), 0)
    acc = jnp.zeros((TB, D), x_full.dtype)
    cls_sel = jnp.zeros((TB, 1), jnp.bool_)
    for b in range(B):
        parts = [x_full, jnp.zeros((B - b, D), x_full.dtype)]
        if b > 0:
            parts.insert(0, jnp.zeros((b, D), x_full.dtype))
        sh = jnp.concatenate(parts, axis=0)
        sel = (o2 >= ptr_ref[b] + b) & (o2 < ptr_ref[b + 1] + b)
        acc = jnp.where(sel, sh, acc)
        cls_sel = cls_sel | (o2 == ptr_ref[b + 1] + b)
    acc = jnp.where(cls_sel, cls_ref[...], acc)
    out_x_ref[...] = acc

    rows, cols = out_nb_ref.shape
    o = (jax.lax.broadcasted_iota(jnp.int32, (rows, cols), 0) * cols
         + jax.lax.broadcasted_iota(jnp.int32, (rows, cols), 1))
    nb = jnp.zeros((rows, cols), jnp.int32)
    cm = jnp.zeros((rows, cols), jnp.bool_)
    for b in range(1, B + 1):
        nb = nb + (o >= ptr_ref[b] + b).astype(jnp.int32)
    for b in range(B):
        cm = cm | (o == ptr_ref[b + 1] + b)
    out_nb_ref[...] = nb
    out_cm_ref[...] = cm


def kernel(x, batch_ids, ptr, edges, cls, cls_edge):
    B, N, _, Ed = edges.shape
    T, D = x.shape
    R = N * Ed
    R1 = R + Ed

    et = jnp.transpose(edges, (0, 1, 3, 2)).reshape(B, R, N)

    clsblk = jnp.broadcast_to(
        cls_edge.reshape(1, 1, Ed, 1), (1, _RB // Ed, Ed, N + 1)
    ).reshape(1, _RB, N + 1)

    nfull = R // _RB
    n_rblocks = (R1 + _RB - 1) // _RB

    out_e, mask = pl.pallas_call(
        functools.partial(_edges_mask_body, N, nfull),
        grid=(B, n_rblocks),
        in_specs=[
            pl.BlockSpec(memory_space=pltpu.SMEM),
            pl.BlockSpec((1, _RB, N), lambda b, r: (b, jnp.minimum(r, nfull - 1), 0)),
            pl.BlockSpec((1, _RB, N + 1), lambda b, r: (0, 0, 0)),
        ],
        out_specs=[
            pl.BlockSpec((1, _RB, N + 1), lambda b, r: (b, r, 0)),
            pl.BlockSpec((1, N + 1, N + 1), lambda b, r: (b, 0, 0)),
        ],
        out_shape=[
            jax.ShapeDtypeStruct((B, R1, N + 1), jnp.float32),
            jax.ShapeDtypeStruct((B, N + 1, N + 1), jnp.int8),
        ],
    )(ptr, et, clsblk)
    mask = mask.astype(jnp.bool_)

    edges_out = out_e.reshape(B, N + 1, Ed, N + 1).transpose(0, 1, 3, 2)

    TB = T + B
    rows = B
    cols = TB // B
    assert rows * cols == TB

    new_x, nb, cm = pl.pallas_call(
        functools.partial(_x_body, B, T),
        in_specs=[
            pl.BlockSpec(memory_space=pltpu.SMEM),
            pl.BlockSpec(memory_space=pltpu.VMEM),
            pl.BlockSpec(memory_space=pltpu.VMEM),
        ],
        out_specs=[
            pl.BlockSpec(memory_space=pltpu.VMEM),
            pl.BlockSpec(memory_space=pltpu.VMEM),
            pl.BlockSpec(memory_space=pltpu.VMEM),
        ],
        out_shape=[
            jax.ShapeDtypeStruct((TB, D), x.dtype),
            jax.ShapeDtypeStruct((rows, cols), jnp.int32),
            jax.ShapeDtypeStruct((rows, cols), jnp.bool_),
        ],
    )(ptr, x, cls.reshape(1, D))

    new_batch = nb.reshape(TB)
    cls_mask = cm.reshape(TB)
    new_ptr = ptr + jnp.arange(B + 1, dtype=ptr.dtype)
    return new_x, mask, edges_out, cls_mask, new_batch, new_ptr


# speedup vs baseline: 9.3710x; 1.1006x over previous
import functools

import jax
import jax.numpy as jnp
from jax.experimental import pallas as pl
from jax.experimental.pallas import tpu as pltpu


_RB = 4096


def _edges_mask_body(N, Ed, nfull, ptr_ref, in_ref, clscol_ref, clsslab_ref,
                     out_e_ref, out_m_ref):
    b = pl.program_id(0)
    r = pl.program_id(1)

    @pl.when(r < nfull)
    def _():
        out_e_ref[:, :, 0:N] = in_ref[...]
        out_e_ref[:, :, N:N + 1] = clscol_ref[:, :, 0:1]

    @pl.when(r == nfull)
    def _():
        out_e_ref[:, 0:Ed, :] = clsslab_ref[...]

    @pl.when(r == 0)
    def _():
        c = ptr_ref[b + 1] - ptr_ref[b]
        rows = jax.lax.broadcasted_iota(jnp.int32, (1, N + 1, 1), 1)
        cols = jax.lax.broadcasted_iota(jnp.int32, (1, 1, N + 1), 2)
        f_rows = (rows < c) | (rows == N)
        f_cols = (cols < c) | (cols == N)
        out_m_ref[...] = (f_rows & f_cols & (c > 0)).astype(jnp.int8)


def _x_body(B, T, ptr_ref, x_ref, cls_ref, out_x_ref, out_nb_ref, out_cm_ref):
    D = x_ref.shape[1]
    TB = T + B
    x_full = x_ref[...]
    o2 = jax.lax.broadcasted_iota(jnp.int32, (TB, 1), 0)
    acc = jnp.zeros((TB, D), x_full.dtype)
    cls_sel = jnp.zeros((TB, 1), jnp.bool_)
    for b in range(B):
        parts = [x_full, jnp.zeros((B - b, D), x_full.dtype)]
        if b > 0:
            parts.insert(0, jnp.zeros((b, D), x_full.dtype))
        sh = jnp.concatenate(parts, axis=0)
        sel = (o2 >= ptr_ref[b] + b) & (o2 < ptr_ref[b + 1] + b)
        acc = jnp.where(sel, sh, acc)
        cls_sel = cls_sel | (o2 == ptr_ref[b + 1] + b)
    acc = jnp.where(cls_sel, cls_ref[...], acc)
    out_x_ref[...] = acc

    rows, cols = out_nb_ref.shape
    o = (jax.lax.broadcasted_iota(jnp.int32, (rows, cols), 0) * cols
         + jax.lax.broadcasted_iota(jnp.int32, (rows, cols), 1))
    nb = jnp.zeros((rows, cols), jnp.int32)
    cm = jnp.zeros((rows, cols), jnp.bool_)
    for b in range(1, B + 1):
        nb = nb + (o >= ptr_ref[b] + b).astype(jnp.int32)
    for b in range(B):
        cm = cm | (o == ptr_ref[b + 1] + b)
    out_nb_ref[...] = nb
    out_cm_ref[...] = cm


def kernel(x, batch_ids, ptr, edges, cls, cls_edge):
    B, N, _, Ed = edges.shape
    T, D = x.shape
    R = N * Ed
    R1 = R + Ed

    et = jnp.transpose(edges, (0, 1, 3, 2)).reshape(B, R, N)

    clscol = jnp.broadcast_to(
        cls_edge.reshape(1, 1, Ed, 1), (1, _RB // Ed, Ed, 128)
    ).reshape(1, _RB, 128)
    clsslab = jnp.broadcast_to(cls_edge.reshape(1, Ed, 1), (1, Ed, N + 1))

    assert R % _RB == 0
    nfull = R // _RB
    n_rblocks = nfull + 1

    out_e, mask = pl.pallas_call(
        functools.partial(_edges_mask_body, N, Ed, nfull),
        grid=(B, n_rblocks),
        in_specs=[
            pl.BlockSpec(memory_space=pltpu.SMEM),
            pl.BlockSpec((1, _RB, N), lambda b, r: (b, jnp.minimum(r, nfull - 1), 0)),
            pl.BlockSpec((1, _RB, 128), lambda b, r: (0, 0, 0)),
            pl.BlockSpec((1, Ed, N + 1), lambda b, r: (0, 0, 0)),
        ],
        out_specs=[
            pl.BlockSpec((1, _RB, N + 1), lambda b, r: (b, r, 0)),
            pl.BlockSpec((1, N + 1, N + 1), lambda b, r: (b, 0, 0)),
        ],
        out_shape=[
            jax.ShapeDtypeStruct((B, R1, N + 1), jnp.float32),
            jax.ShapeDtypeStruct((B, N + 1, N + 1), jnp.int8),
        ],
    )(ptr, et, clscol, clsslab)
    mask = mask.astype(jnp.bool_)

    edges_out = out_e.reshape(B, N + 1, Ed, N + 1).transpose(0, 1, 3, 2)

    TB = T + B
    rows = B
    cols = TB // B
    assert rows * cols == TB

    new_x, nb, cm = pl.pallas_call(
        functools.partial(_x_body, B, T),
        in_specs=[
            pl.BlockSpec(memory_space=pltpu.SMEM),
            pl.BlockSpec(memory_space=pltpu.VMEM),
            pl.BlockSpec(memory_space=pltpu.VMEM),
        ],
        out_specs=[
            pl.BlockSpec(memory_space=pltpu.VMEM),
            pl.BlockSpec(memory_space=pltpu.VMEM),
            pl.BlockSpec(memory_space=pltpu.VMEM),
        ],
        out_shape=[
            jax.ShapeDtypeStruct((TB, D), x.dtype),
            jax.ShapeDtypeStruct((rows, cols), jnp.int32),
            jax.ShapeDtypeStruct((rows, cols), jnp.bool_),
        ],
    )(ptr, x, cls.reshape(1, D))

    new_batch = nb.reshape(TB)
    cls_mask = cm.reshape(TB)
    new_ptr = ptr + jnp.arange(B + 1, dtype=ptr.dtype)
    return new_x, mask, edges_out, cls_mask, new_batch, new_ptr
